# baseline (speedup 1.0000x reference)
import functools

import jax
import jax.numpy as jnp
from jax import lax
from jax.experimental import pallas as pl
from jax.experimental.pallas import tpu as pltpu
from jax.experimental.pallas import tpu_sc as plsc

NC, NS, L = 2, 16, 16
NW = NC * NS
NCOL = 26
BATCH = 16384
LOGB = 14
DIM = 64
WDIM = 2 * DIM
SEG = 100000
NROWS = NCOL * SEG
FLAT = BATCH * NCOL
PER_W = FLAT // NW
CH = 832
NCHUNK = PER_W // CH


def _build():
    mesh = plsc.VectorSubcoreMesh(
        core_axis_name="c", subcore_axis_name="s",
        num_cores=NC, num_subcores=NS,
    )

    @functools.partial(
        pl.kernel,
        out_type=jax.ShapeDtypeStruct((FLAT, WDIM), jnp.bfloat16),
        mesh=mesh,
        compiler_params=pltpu.CompilerParams(use_tc_tiling_on_sc=False),
        scratch_types=[
            pltpu.VMEM((PER_W,), jnp.int32),
            pltpu.VMEM((2, CH, WDIM), jnp.bfloat16),
            pltpu.SemaphoreType.DMA((2,)),
            pltpu.SemaphoreType.DMA((2,)),
        ],
    )
    def k(x_hbm, table_hbm, out_hbm, idx_v, rows, gsem, ssem):
        wid = lax.axis_index("c") * NS + lax.axis_index("s")
        base = wid * PER_W

        pltpu.sync_copy(x_hbm.at[pl.ds(base, PER_W)], idx_v)

        lane = jax.lax.iota(jnp.int32, L)

        @pl.loop(0, PER_W // L)
        def _add_offsets(j):
            sl = pl.ds(j * L, L)
            pos = lane + (base + j * L)
            idx_v[sl] = idx_v[sl] + (pos >> LOGB) * SEG

        def gather(j, b):
            return pltpu.make_async_copy(
                table_hbm.at[idx_v.at[pl.ds(j * CH, CH)]], rows.at[b],
                gsem.at[b])

        def scatter(j, b):
            return pltpu.make_async_copy(
                rows.at[b], out_hbm.at[pl.ds(base + j * CH, CH)], ssem.at[b])

        gather(0, 0).start()
        for j in range(NCHUNK):
            b = j & 1
            gather(j, b).wait()
            if j + 1 < NCHUNK:
                if j >= 1:
                    scatter(j - 1, 1 - b).wait()
                gather(j + 1, 1 - b).start()
            scatter(j, b).start()
        scatter(NCHUNK - 2, (NCHUNK - 2) & 1).wait()
        scatter(NCHUNK - 1, (NCHUNK - 1) & 1).wait()

    return k


_lookup = _build()


def kernel(x_categ, table):
    x_cm = x_categ.astype(jnp.int32).T.reshape(FLAT)
    table_w = jax.lax.bitcast_convert_type(table, jnp.bfloat16).reshape(NROWS, WDIM)
    out = _lookup(x_cm, table_w)
    out_f32 = jax.lax.bitcast_convert_type(out.reshape(FLAT, DIM, 2), jnp.float32)
    return out_f32.reshape(NCOL, BATCH, DIM).transpose(1, 0, 2)


# speedup vs baseline: 4.5400x; 4.5400x over previous
import functools

import jax
import jax.numpy as jnp
from jax import lax
from jax.experimental import pallas as pl
from jax.experimental.pallas import tpu as pltpu
from jax.experimental.pallas import tpu_sc as plsc

NC, NS, L = 2, 16, 16
NW = NC * NS
NCOL = 26
BATCH = 16384
LOGB = 14
DIM = 64
HDIM = DIM // 2
SEG = 100000
NROWS = NCOL * SEG
FLAT = BATCH * NCOL
PER_W = FLAT // NW
PER_W2 = 2 * PER_W
CHH = 832
NCHUNK = PER_W2 // CHH


def _build():
    mesh = plsc.VectorSubcoreMesh(
        core_axis_name="c", subcore_axis_name="s",
        num_cores=NC, num_subcores=NS,
    )

    @functools.partial(
        pl.kernel,
        out_type=jax.ShapeDtypeStruct((2 * FLAT, HDIM), jnp.float32),
        mesh=mesh,
        compiler_params=pltpu.CompilerParams(
            use_tc_tiling_on_sc=False, needs_layout_passes=False),
        scratch_types=[
            pltpu.VMEM((PER_W,), jnp.int32),
            pltpu.VMEM((PER_W2,), jnp.int32),
            pltpu.VMEM((2, CHH, HDIM), jnp.float32),
            pltpu.SemaphoreType.DMA((2,)),
            pltpu.SemaphoreType.DMA((2,)),
        ],
    )
    def k(x_hbm, table_hbm, out_hbm, idx_v, idx2_v, rows, gsem, ssem):
        wid = lax.axis_index("c") * NS + lax.axis_index("s")
        base = wid * PER_W

        pltpu.sync_copy(x_hbm.at[pl.ds(base, PER_W)], idx_v)

        lane = jax.lax.iota(jnp.int32, L)

        @pl.loop(0, PER_W2 // L)
        def _expand(j):
            qv = lane + j * L
            pv = qv >> 1
            xv = plsc.load_gather(idx_v, [pv])
            off = ((base + pv) >> LOGB) * SEG
            idx2_v[pl.ds(j * L, L)] = ((xv + off) << 1) | (qv & 1)

        def gather(j, b):
            return pltpu.make_async_copy(
                table_hbm.at[idx2_v.at[pl.ds(j * CHH, CHH)]], rows.at[b],
                gsem.at[b])

        def scatter(j, b):
            return pltpu.make_async_copy(
                rows.at[b],
                out_hbm.at[pl.ds(2 * base + j * CHH, CHH)], ssem.at[b])

        gather(0, 0).start()
        for j in range(NCHUNK):
            b = j & 1
            gather(j, b).wait()
            if j + 1 < NCHUNK:
                if j >= 1:
                    scatter(j - 1, 1 - b).wait()
                gather(j + 1, 1 - b).start()
            scatter(j, b).start()
        scatter(NCHUNK - 2, (NCHUNK - 2) & 1).wait()
        scatter(NCHUNK - 1, (NCHUNK - 1) & 1).wait()

    return k


_lookup = _build()


def kernel(x_categ, table):
    x_cm = x_categ.astype(jnp.int32).T.reshape(FLAT)
    table_h = table.reshape(2 * NROWS, HDIM)
    out = _lookup(x_cm, table_h)
    return out.reshape(NCOL, BATCH, DIM).transpose(1, 0, 2)


# speedup vs baseline: 4.5659x; 1.0057x over previous
import functools

import jax
import jax.numpy as jnp
from jax import lax
from jax.experimental import pallas as pl
from jax.experimental.pallas import tpu as pltpu
from jax.experimental.pallas import tpu_sc as plsc

NC, NS, L = 2, 16, 16
NW = NC * NS
NCOL = 26
BATCH = 16384
LOGB = 14
DIM = 64
SEG = 100000
FLAT = BATCH * NCOL
PER_W = FLAT // NW
CH = 832
NCHUNK = PER_W // CH


def _build():
    mesh = plsc.VectorSubcoreMesh(
        core_axis_name="c", subcore_axis_name="s",
        num_cores=NC, num_subcores=NS,
    )

    @functools.partial(
        pl.kernel,
        out_type=jax.ShapeDtypeStruct((FLAT, DIM), jnp.float32),
        mesh=mesh,
        compiler_params=pltpu.CompilerParams(use_tc_tiling_on_sc=False),
        scratch_types=[
            pltpu.VMEM((PER_W,), jnp.int32),
            pltpu.VMEM((2, CH, DIM), jnp.float32),
            pltpu.SemaphoreType.DMA((2,)),
            pltpu.SemaphoreType.DMA((2,)),
        ],
    )
    def k(x_hbm, table_hbm, out_hbm, idx_v, rows, gsem, ssem):
        wid = lax.axis_index("c") * NS + lax.axis_index("s")
        base = wid * PER_W

        pltpu.sync_copy(x_hbm.at[pl.ds(base, PER_W)], idx_v)

        lane = jax.lax.iota(jnp.int32, L)

        @pl.loop(0, PER_W // L)
        def _add_offsets(j):
            sl = pl.ds(j * L, L)
            pos = lane + (base + j * L)
            idx_v[sl] = idx_v[sl] + (pos >> LOGB) * SEG

        def gather(j, b):
            return pltpu.make_async_copy(
                table_hbm.at[idx_v.at[pl.ds(j * CH, CH)]], rows.at[b],
                gsem.at[b])

        def scatter(j, b):
            return pltpu.make_async_copy(
                rows.at[b], out_hbm.at[pl.ds(base + j * CH, CH)], ssem.at[b])

        gather(0, 0).start()
        for j in range(NCHUNK):
            b = j & 1
            gather(j, b).wait()
            if j + 1 < NCHUNK:
                if j >= 1:
                    scatter(j - 1, 1 - b).wait()
                gather(j + 1, 1 - b).start()
            scatter(j, b).start()
        scatter(NCHUNK - 2, (NCHUNK - 2) & 1).wait()
        scatter(NCHUNK - 1, (NCHUNK - 1) & 1).wait()

    return k


_lookup = _build()


def kernel(x_categ, table):
    x_cm = x_categ.astype(jnp.int32).T.reshape(FLAT)
    out = _lookup(x_cm, table)
    return out.reshape(NCOL, BATCH, DIM).transpose(1, 0, 2)
